# baseline (speedup 1.0000x reference)
import functools

import jax
import jax.numpy as jnp
from jax import lax
from jax.experimental import pallas as pl
from jax.experimental.pallas import tpu as pltpu, tpu_sc as plsc

B = 1024
T = 200
D = 128

_info = plsc.get_sparse_core_info()
_NC, _NS, _L = _info.num_cores, _info.num_subcores, _info.num_lanes
_NW = _NC * _NS
_BPW = B // _NW
_GROUPS = _BPW // _L


def _eos_gather_body(x_hbm, text_hbm, out_hbm, text_v, idx_v, rows_v, sem):
    wid = lax.axis_index("s") * _NC + lax.axis_index("c")
    base = wid * _BPW

    pltpu.sync_copy(text_hbm.at[pl.ds(base * T, _BPW * T)], text_v)

    lane = lax.iota(jnp.int32, _L)
    for g in range(_GROUPS):
        row_off = (jnp.full((_L,), g * _L, jnp.int32) + lane) * T

        cnt = jnp.zeros((_L,), jnp.int32)
        for j in range(T):
            v = plsc.load_gather(text_v, [row_off + j])
            cnt = cnt + (v != 0).astype(jnp.int32)
        eos = jnp.clip(cnt - 1, 0, T - 1)
        flat = (jnp.full((_L,), base + g * _L, jnp.int32) + lane) * T + eos
        idx_v[pl.ds(g * _L, _L)] = flat

    pltpu.async_copy(x_hbm.at[idx_v], rows_v, sem).wait()
    pltpu.sync_copy(rows_v, out_hbm.at[pl.ds(base, _BPW)])


@jax.jit
def kernel(x, text):
    x2 = x.reshape(B * T, D)
    text32 = text.astype(jnp.int32).reshape(B * T)
    mesh = plsc.VectorSubcoreMesh(core_axis_name="c", subcore_axis_name="s")
    run = functools.partial(
        pl.kernel,
        mesh=mesh,
        compiler_params=pltpu.CompilerParams(
            needs_layout_passes=False,
            skip_device_barrier=True,
            disable_bounds_checks=True,
            disable_semaphore_checks=True,
        ),
        out_type=jax.ShapeDtypeStruct((B, D), jnp.float32),
        scratch_types=[
            pltpu.VMEM((_BPW * T,), jnp.int32),
            pltpu.VMEM((_BPW,), jnp.int32),
            pltpu.VMEM((_BPW, D), jnp.float32),
            pltpu.SemaphoreType.DMA,
        ],
    )(_eos_gather_body)
    return run(x2, text32)


# speedup vs baseline: 2.2655x; 2.2655x over previous
import jax
import jax.numpy as jnp
from jax import lax
from jax.experimental import pallas as pl
from jax.experimental.pallas import tpu as pltpu

B = 1024
T = 200
D = 128
_UNROLL = 8


def _eos_gather_body(x_hbm, text_ref, out_ref, flat_v, flat_s, sem0, sem1):
    t = text_ref[...]
    cnt = jnp.sum((t != 0).astype(jnp.int32), axis=1)
    eos = jnp.clip(cnt - 1, 0, T - 1)
    flat_v[...] = lax.broadcasted_iota(jnp.int32, (B,), 0) * T + eos

    pltpu.make_async_copy(flat_v, flat_s, sem0).start()
    pltpu.make_async_copy(flat_v, flat_s, sem0).wait()

    def issue(i, carry):
        for u in range(_UNROLL):
            ii = i * _UNROLL + u
            r = flat_s[ii]
            pltpu.make_async_copy(
                x_hbm.at[pl.ds(r, 1)], out_ref.at[pl.ds(ii, 1)], sem1
            ).start()
        return carry

    lax.fori_loop(0, B // _UNROLL, issue, 0)
    pltpu.make_async_copy(x_hbm.at[pl.ds(0, B)], out_ref, sem1).wait()


@jax.jit
def kernel(x, text):
    x2 = x.reshape(B * T, D)
    text32 = text.astype(jnp.int32)
    return pl.pallas_call(
        _eos_gather_body,
        in_specs=[
            pl.BlockSpec(memory_space=pl.ANY),
            pl.BlockSpec(memory_space=pltpu.VMEM),
        ],
        out_specs=pl.BlockSpec(memory_space=pltpu.VMEM),
        out_shape=jax.ShapeDtypeStruct((B, D), jnp.float32),
        scratch_shapes=[
            pltpu.VMEM((B,), jnp.int32),
            pltpu.SMEM((B,), jnp.int32),
            pltpu.SemaphoreType.DMA,
            pltpu.SemaphoreType.DMA,
        ],
    )(x2, text32)


# speedup vs baseline: 2.6453x; 1.1676x over previous
import jax
import jax.numpy as jnp
from jax import lax
from jax.experimental import pallas as pl
from jax.experimental.pallas import tpu as pltpu

B = 1024
T = 200
D = 128
_UNROLL = 8


def _eos_gather_body(x_hbm, text_ref, out_ref, flat_v, flat_s, sem0, sem1):
    t = text_ref[...]
    cnt = jnp.sum((t != 0).astype(jnp.int32), axis=1)
    eos = jnp.clip(cnt - 1, 0, T - 1)
    flat_v[...] = lax.broadcasted_iota(jnp.int32, (B,), 0) * T + eos

    pltpu.make_async_copy(flat_v, flat_s, sem0).start()
    pltpu.make_async_copy(flat_v, flat_s, sem0).wait()

    def issue(i, carry):
        for u in range(_UNROLL):
            ii = i * _UNROLL + u
            r = flat_s[ii]
            pltpu.make_async_copy(
                x_hbm.at[pl.ds(r, 1)], out_ref.at[pl.ds(ii, 1)], sem1
            ).start(priority=u % 2)
        return carry

    lax.fori_loop(0, B // _UNROLL, issue, 0)
    pltpu.make_async_copy(x_hbm.at[pl.ds(0, B)], out_ref, sem1).wait()


@jax.jit
def kernel(x, text):
    x2 = x.reshape(B * T, D)
    text32 = text.astype(jnp.int32)
    return pl.pallas_call(
        _eos_gather_body,
        in_specs=[
            pl.BlockSpec(memory_space=pl.ANY),
            pl.BlockSpec(memory_space=pltpu.VMEM),
        ],
        out_specs=pl.BlockSpec(memory_space=pltpu.VMEM),
        out_shape=jax.ShapeDtypeStruct((B, D), jnp.float32),
        scratch_shapes=[
            pltpu.VMEM((B,), jnp.int32),
            pltpu.SMEM((B,), jnp.int32),
            pltpu.SemaphoreType.DMA,
            pltpu.SemaphoreType.DMA,
        ],
    )(x2, text32)
